# baseline (speedup 1.0000x reference)
import functools

import jax
import jax.numpy as jnp
from jax import lax
from jax.experimental import pallas as pl
from jax.experimental.pallas import tpu as pltpu
from jax.experimental.pallas import tpu_sc as plsc

N = 10000
E = 320000
D = 128
D_OUT = 64

NC = 2
NS = 16
NW = NC * NS
CH = 128
EPW = 79 * CH
EPAD = NW * EPW
PN = 10240
RP = PN // NS

_sc_mesh = plsc.VectorSubcoreMesh(core_axis_name="c", subcore_axis_name="s")


@functools.partial(
    pl.kernel,
    out_type=jax.ShapeDtypeStruct((NC * PN, D), jnp.float32),
    mesh=_sc_mesh,
    scratch_types=[
        pltpu.VMEM((CH, D), jnp.float32),
        pltpu.VMEM((CH,), jnp.int32),
        pltpu.VMEM((1, CH), jnp.int32),
        pltpu.SemaphoreType.DMA,
        pltpu.VMEM_SHARED((PN, D), jnp.float32),
    ],
)
def _sc_prop(t_hbm, src_hbm, dst_hbm, out_hbm, rows, sidx, didx, gsem, acc):
    c = lax.axis_index("c")
    s = lax.axis_index("s")
    wid = c * NS + s

    def initc(j, carry):
        r0 = pl.multiple_of(s * RP + j * CH, 8)
        pltpu.sync_copy(t_hbm.at[pl.ds(r0, CH)], rows)
        pltpu.sync_copy(rows, acc.at[pl.ds(r0, CH)])
        return carry

    lax.fori_loop(0, RP // CH, initc, 0)
    plsc.subcore_barrier()

    def chunk(j, carry):
        e0 = pl.multiple_of(wid * EPW + j * CH, 8)
        pltpu.sync_copy(src_hbm.at[pl.ds(e0, CH)], sidx)
        pltpu.sync_copy(dst_hbm.at[pl.ds(e0, CH)], didx.at[0])
        pltpu.async_copy(t_hbm.at[sidx], rows, gsem).wait()
        pltpu.sync_copy(rows, acc.at[didx.at[0]], add=True)
        return carry

    lax.fori_loop(0, EPW // CH, chunk, 0)
    plsc.subcore_barrier()

    def wbc(j, carry):
        r0 = pl.multiple_of(s * RP + j * CH, 8)
        pltpu.sync_copy(acc.at[pl.ds(r0, CH)], rows)
        pltpu.sync_copy(rows, out_hbm.at[pl.ds(c * PN + r0, CH)])
        return carry

    lax.fori_loop(0, RP // CH, wbc, 0)


BR = 1000


def _tc0_body(x_ref, w_ref, xw_ref):
    xw_ref[...] = jnp.dot(x_ref[...], w_ref[...],
                          preferred_element_type=jnp.float32)


def _tc_mid_body(p_ref, t_ref, dis_ref, xw_ref, b_ref, fa_ref, fb_ref,
                 o_ref, dout_ref):
    comb = p_ref[0] + p_ref[1] - t_ref[...]
    is_first = fa_ref[...] > 0.0
    relu_on = fb_ref[...] > 0.0
    d = jnp.where(is_first[:, 0:1], lax.rsqrt(comb[:, 0:1]), dis_ref[...])
    zc = d * comb + b_ref[...]
    g = jnp.where(relu_on, jnp.maximum(zc, 0.0), zc)
    o_ref[...] = jnp.where(is_first, d * xw_ref[...],
                           jnp.where(relu_on, d * g, g))
    dout_ref[...] = d


def _tc3_body(hp_ref, wm_ref, bm_ref, wl_ref, bl_ref, mu_ref, lv_ref):
    hp = hp_ref[...]
    mu_ref[...] = jnp.dot(hp, wm_ref[...],
                          preferred_element_type=jnp.float32) + bm_ref[...]
    lv_ref[...] = jnp.dot(hp, wl_ref[...],
                          preferred_element_type=jnp.float32) + bl_ref[...]


def _row_spec(width):
    return pl.BlockSpec((BR, width), lambda i: (i, 0))


_pq_spec = pl.BlockSpec((2, BR, D), lambda i: (0, i, 0))


def _full_spec(shape):
    nd = len(shape)
    return pl.BlockSpec(shape, lambda i: (0,) * nd)


_tc0 = pl.pallas_call(
    _tc0_body,
    grid=(N // BR,),
    in_specs=[_row_spec(D), _full_spec((D, D))],
    out_specs=_row_spec(D),
    out_shape=jax.ShapeDtypeStruct((PN, D), jnp.float32),
)

_tc_mid = pl.pallas_call(
    _tc_mid_body,
    grid=(N // BR,),
    in_specs=[_pq_spec, _row_spec(D), _row_spec(1), _row_spec(D),
              _full_spec((1, D)), _full_spec((1, D)), _full_spec((1, D))],
    out_specs=[_row_spec(D), _row_spec(1)],
    out_shape=[jax.ShapeDtypeStruct((PN, D), jnp.float32),
               jax.ShapeDtypeStruct((PN, 1), jnp.float32)],
)

_tc3 = pl.pallas_call(
    _tc3_body,
    grid=(N // BR,),
    in_specs=[_row_spec(D),
              _full_spec((D, D_OUT)), _full_spec((1, D_OUT)),
              _full_spec((D, D_OUT)), _full_spec((1, D_OUT))],
    out_specs=[_row_spec(D_OUT), _row_spec(D_OUT)],
    out_shape=[jax.ShapeDtypeStruct((N, D_OUT), jnp.float32),
               jax.ShapeDtypeStruct((N, D_OUT), jnp.float32)],
)


@jax.jit
def kernel(x, edge_index, W1, b1, W_mu, b_mu, W_logvar, b_logvar):
    ei = edge_index.astype(jnp.int32)
    npad = EPAD - E
    src = jnp.concatenate([ei[0], jnp.zeros((npad,), jnp.int32)])
    dst = jnp.concatenate([ei[1], jnp.full((npad,), PN - 1, jnp.int32)])

    xw = _tc0(x, W1)

    t0 = jnp.ones((PN, D), jnp.float32)
    dis0 = jnp.ones((PN, 1), jnp.float32)
    zerosD = jnp.zeros((1, D), jnp.float32)
    onesD = jnp.ones((1, D), jnp.float32)

    niter = 3 + lax.optimization_barrier(jnp.int32(0))

    def layer(i, carry):
        t, dis = carry
        fa = jnp.where(i == 0, onesD, zerosD)
        fb = jnp.where(i == 1, onesD, zerosD)
        b_i = jnp.where(i == 1, b1.reshape(1, D), zerosD)
        p = _sc_prop(t, src, dst).reshape(NC, PN, D)
        t2, dis2 = _tc_mid(p, t, dis, xw, b_i, fa, fb)
        return (t2, dis2)

    hp, _ = lax.fori_loop(0, niter, layer, (t0, dis0))

    mu, logvar = _tc3(hp,
                      W_mu, b_mu.reshape(1, D_OUT),
                      W_logvar, b_logvar.reshape(1, D_OUT))
    return (mu, logvar)


# speedup vs baseline: 1.0406x; 1.0406x over previous
import functools

import jax
import jax.numpy as jnp
from jax import lax
from jax.experimental import pallas as pl
from jax.experimental.pallas import tpu as pltpu
from jax.experimental.pallas import tpu_sc as plsc

N = 10000
E = 320000
D = 128
D_OUT = 64

NC = 2
NS = 16
NW = NC * NS
CH = 96
EPW = 105 * CH
EPAD = NW * EPW
PN = 10240
RP = PN // NS

_sc_mesh = plsc.VectorSubcoreMesh(core_axis_name="c", subcore_axis_name="s")


@functools.partial(
    pl.kernel,
    out_type=jax.ShapeDtypeStruct((NC * PN, D), jnp.float32),
    mesh=_sc_mesh,
    scratch_types=[
        pltpu.VMEM((CH, D), jnp.float32),
        pltpu.VMEM((CH,), jnp.int32),
        pltpu.VMEM((1, CH), jnp.int32),
        pltpu.SemaphoreType.DMA,
        pltpu.VMEM_SHARED((PN, D), jnp.float32),
    ],
)
def _sc_prop(t_hbm, src_hbm, dst_hbm, out_hbm, rows, sidx, didx, gsem, acc):
    c = lax.axis_index("c")
    s = lax.axis_index("s")
    wid = c * NS + s

    def initc(j, carry):
        r0 = pl.multiple_of(s * RP + j * CH, 8)
        pltpu.sync_copy(t_hbm.at[pl.ds(r0, CH)], rows)
        pltpu.sync_copy(rows, acc.at[pl.ds(r0, CH)])
        return carry

    lax.fori_loop(0, RP // CH, initc, 0)
    plsc.subcore_barrier()

    def chunk(j, carry):
        e0 = pl.multiple_of(wid * EPW + j * CH, 8)
        pltpu.sync_copy(src_hbm.at[pl.ds(e0, CH)], sidx)
        pltpu.sync_copy(dst_hbm.at[pl.ds(e0, CH)], didx.at[0])
        pltpu.async_copy(t_hbm.at[sidx], rows, gsem).wait()
        pltpu.sync_copy(rows, acc.at[didx.at[0]], add=True)
        return carry

    lax.fori_loop(0, EPW // CH, chunk, 0)
    plsc.subcore_barrier()

    def wbc(j, carry):
        r0 = pl.multiple_of(s * RP + j * CH, 8)
        pltpu.sync_copy(acc.at[pl.ds(r0, CH)], rows)
        pltpu.sync_copy(rows, out_hbm.at[pl.ds(c * PN + r0, CH)])
        return carry

    lax.fori_loop(0, RP // CH, wbc, 0)


BR = 1000


def _tc0_body(x_ref, w_ref, xw_ref):
    xw_ref[...] = jnp.dot(x_ref[...], w_ref[...],
                          preferred_element_type=jnp.float32)


def _tc_mid_body(p_ref, t_ref, dis_ref, xw_ref, b_ref, fa_ref, fb_ref,
                 o_ref, dout_ref):
    comb = p_ref[0] + p_ref[1] - t_ref[...]
    is_first = fa_ref[...] > 0.0
    relu_on = fb_ref[...] > 0.0
    d = jnp.where(is_first[:, 0:1], lax.rsqrt(comb[:, 0:1]), dis_ref[...])
    zc = d * comb + b_ref[...]
    g = jnp.where(relu_on, jnp.maximum(zc, 0.0), zc)
    o_ref[...] = jnp.where(is_first, d * xw_ref[...],
                           jnp.where(relu_on, d * g, g))
    dout_ref[...] = d


def _tc3_body(hp_ref, wm_ref, bm_ref, wl_ref, bl_ref, mu_ref, lv_ref):
    hp = hp_ref[...]
    mu_ref[...] = jnp.dot(hp, wm_ref[...],
                          preferred_element_type=jnp.float32) + bm_ref[...]
    lv_ref[...] = jnp.dot(hp, wl_ref[...],
                          preferred_element_type=jnp.float32) + bl_ref[...]


def _row_spec(width):
    return pl.BlockSpec((BR, width), lambda i: (i, 0))


_pq_spec = pl.BlockSpec((2, BR, D), lambda i: (0, i, 0))


def _full_spec(shape):
    nd = len(shape)
    return pl.BlockSpec(shape, lambda i: (0,) * nd)


_tc0 = pl.pallas_call(
    _tc0_body,
    grid=(N // BR,),
    in_specs=[_row_spec(D), _full_spec((D, D))],
    out_specs=_row_spec(D),
    out_shape=jax.ShapeDtypeStruct((PN, D), jnp.float32),
)

_tc_mid = pl.pallas_call(
    _tc_mid_body,
    grid=(N // BR,),
    in_specs=[_pq_spec, _row_spec(D), _row_spec(1), _row_spec(D),
              _full_spec((1, D)), _full_spec((1, D)), _full_spec((1, D))],
    out_specs=[_row_spec(D), _row_spec(1)],
    out_shape=[jax.ShapeDtypeStruct((PN, D), jnp.float32),
               jax.ShapeDtypeStruct((PN, 1), jnp.float32)],
)

_tc3 = pl.pallas_call(
    _tc3_body,
    grid=(N // BR,),
    in_specs=[_row_spec(D),
              _full_spec((D, D_OUT)), _full_spec((1, D_OUT)),
              _full_spec((D, D_OUT)), _full_spec((1, D_OUT))],
    out_specs=[_row_spec(D_OUT), _row_spec(D_OUT)],
    out_shape=[jax.ShapeDtypeStruct((N, D_OUT), jnp.float32),
               jax.ShapeDtypeStruct((N, D_OUT), jnp.float32)],
)


@jax.jit
def kernel(x, edge_index, W1, b1, W_mu, b_mu, W_logvar, b_logvar):
    ei = edge_index.astype(jnp.int32)
    npad = EPAD - E
    src = jnp.concatenate([ei[0], jnp.zeros((npad,), jnp.int32)])
    dst = jnp.concatenate([ei[1], jnp.full((npad,), PN - 1, jnp.int32)])

    xw = _tc0(x, W1)

    t0 = jnp.ones((PN, D), jnp.float32)
    dis0 = jnp.ones((PN, 1), jnp.float32)
    zerosD = jnp.zeros((1, D), jnp.float32)
    onesD = jnp.ones((1, D), jnp.float32)

    niter = 3 + lax.optimization_barrier(jnp.int32(0))

    def layer(i, carry):
        t, dis = carry
        fa = jnp.where(i == 0, onesD, zerosD)
        fb = jnp.where(i == 1, onesD, zerosD)
        b_i = jnp.where(i == 1, b1.reshape(1, D), zerosD)
        p = _sc_prop(t, src, dst).reshape(NC, PN, D)
        t2, dis2 = _tc_mid(p, t, dis, xw, b_i, fa, fb)
        return (t2, dis2)

    hp, _ = lax.fori_loop(0, niter, layer, (t0, dis0))

    mu, logvar = _tc3(hp,
                      W_mu, b_mu.reshape(1, D_OUT),
                      W_logvar, b_logvar.reshape(1, D_OUT))
    return (mu, logvar)


# speedup vs baseline: 2.1725x; 2.0877x over previous
import functools

import jax
import jax.numpy as jnp
from jax import lax
from jax.experimental import pallas as pl
from jax.experimental.pallas import tpu as pltpu
from jax.experimental.pallas import tpu_sc as plsc

N = 10000
E = 320000
D = 128
D_OUT = 64

NC = 2
NS = 16
NW = NC * NS
CH = 80
EPW = 125 * CH
EPAD = NW * EPW + CH
PN = 10240
RP = PN // NS

_sc_mesh = plsc.VectorSubcoreMesh(core_axis_name="c", subcore_axis_name="s")


@functools.partial(
    pl.kernel,
    out_type=jax.ShapeDtypeStruct((NC * PN, D), jnp.float32),
    mesh=_sc_mesh,
    scratch_types=[
        pltpu.VMEM((CH, D), jnp.float32),
        pltpu.VMEM((CH, D), jnp.float32),
        pltpu.VMEM((CH,), jnp.int32),
        pltpu.VMEM((CH,), jnp.int32),
        pltpu.VMEM((1, CH), jnp.int32),
        pltpu.VMEM((1, CH), jnp.int32),
        pltpu.SemaphoreType.DMA,
        pltpu.SemaphoreType.DMA,
        pltpu.SemaphoreType.DMA,
        pltpu.VMEM_SHARED((PN, D), jnp.float32),
    ],
)
def _sc_prop(t_hbm, src_hbm, dst_hbm, out_hbm,
             rows0, rows1, sidx0, sidx1, didx0, didx1,
             gsem, ssem, isem, acc):
    c = lax.axis_index("c")
    s = lax.axis_index("s")
    wid = c * NS + s
    ebase = wid * EPW

    def e0_of(j):
        return pl.multiple_of(ebase + j * CH, 8)

    def idx_start(j, sb, db):
        pltpu.async_copy(src_hbm.at[pl.ds(e0_of(j), CH)], sb, isem)
        pltpu.async_copy(dst_hbm.at[pl.ds(e0_of(j), CH)], db.at[0], isem)

    def idx_wait(j, sb, db):
        pltpu.make_async_copy(src_hbm.at[pl.ds(e0_of(j), CH)], sb, isem).wait()
        pltpu.make_async_copy(dst_hbm.at[pl.ds(e0_of(j), CH)], db.at[0],
                              isem).wait()

    def gather_start(sb, rw):
        pltpu.async_copy(t_hbm.at[sb], rw, gsem)

    def gather_wait(sb, rw):
        pltpu.make_async_copy(t_hbm.at[sb], rw, gsem).wait()

    def scat_start(rw, db):
        pltpu.async_copy(rw, acc.at[db.at[0]], ssem, add=True)

    def scat_wait(rw, db):
        pltpu.make_async_copy(rw, acc.at[db.at[0]], ssem).wait()

    def initc(j, carry):
        r0 = pl.multiple_of(s * RP + j * CH, 8)
        pltpu.sync_copy(t_hbm.at[pl.ds(r0, CH)], rows0)
        pltpu.sync_copy(rows0, acc.at[pl.ds(r0, CH)])
        return carry

    lax.fori_loop(0, RP // CH, initc, 0)
    plsc.subcore_barrier()

    idx_start(0, sidx0, didx0)
    idx_wait(0, sidx0, didx0)
    gather_start(sidx0, rows0)
    idx_start(1, sidx1, didx1)

    NPAIR = (EPW // CH - 1) // 2

    def pair(k, carry):
        j = k * 2
        gather_wait(sidx0, rows0)
        scat_start(rows0, didx0)
        idx_wait(j + 1, sidx1, didx1)
        gather_start(sidx1, rows1)
        scat_wait(rows0, didx0)
        idx_start(j + 2, sidx0, didx0)
        gather_wait(sidx1, rows1)
        scat_start(rows1, didx1)
        idx_wait(j + 2, sidx0, didx0)
        gather_start(sidx0, rows0)
        scat_wait(rows1, didx1)
        idx_start(j + 3, sidx1, didx1)
        return carry

    lax.fori_loop(0, NPAIR, pair, 0)

    gather_wait(sidx0, rows0)
    scat_start(rows0, didx0)
    idx_wait(2 * NPAIR + 1, sidx1, didx1)
    scat_wait(rows0, didx0)

    plsc.subcore_barrier()

    def wbc(j, carry):
        r0 = pl.multiple_of(s * RP + j * CH, 8)
        pltpu.sync_copy(acc.at[pl.ds(r0, CH)], rows0)
        pltpu.sync_copy(rows0, out_hbm.at[pl.ds(c * PN + r0, CH)])
        return carry

    lax.fori_loop(0, RP // CH, wbc, 0)


BR = 1000


def _tc0_body(x_ref, w_ref, xw_ref):
    xw_ref[...] = jnp.dot(x_ref[...], w_ref[...],
                          preferred_element_type=jnp.float32)


def _tc_mid_body(p_ref, t_ref, dis_ref, xw_ref, b_ref, fa_ref, fb_ref,
                 o_ref, dout_ref):
    comb = p_ref[0] + p_ref[1] - t_ref[...]
    is_first = fa_ref[...] > 0.0
    relu_on = fb_ref[...] > 0.0
    d = jnp.where(is_first[:, 0:1], lax.rsqrt(comb[:, 0:1]), dis_ref[...])
    zc = d * comb + b_ref[...]
    g = jnp.where(relu_on, jnp.maximum(zc, 0.0), zc)
    o_ref[...] = jnp.where(is_first, d * xw_ref[...],
                           jnp.where(relu_on, d * g, g))
    dout_ref[...] = d


def _tc3_body(hp_ref, wm_ref, bm_ref, wl_ref, bl_ref, mu_ref, lv_ref):
    hp = hp_ref[...]
    mu_ref[...] = jnp.dot(hp, wm_ref[...],
                          preferred_element_type=jnp.float32) + bm_ref[...]
    lv_ref[...] = jnp.dot(hp, wl_ref[...],
                          preferred_element_type=jnp.float32) + bl_ref[...]


def _row_spec(width):
    return pl.BlockSpec((BR, width), lambda i: (i, 0))


_pq_spec = pl.BlockSpec((2, BR, D), lambda i: (0, i, 0))


def _full_spec(shape):
    nd = len(shape)
    return pl.BlockSpec(shape, lambda i: (0,) * nd)


_tc0 = pl.pallas_call(
    _tc0_body,
    grid=(N // BR,),
    in_specs=[_row_spec(D), _full_spec((D, D))],
    out_specs=_row_spec(D),
    out_shape=jax.ShapeDtypeStruct((PN, D), jnp.float32),
)

_tc_mid = pl.pallas_call(
    _tc_mid_body,
    grid=(N // BR,),
    in_specs=[_pq_spec, _row_spec(D), _row_spec(1), _row_spec(D),
              _full_spec((1, D)), _full_spec((1, D)), _full_spec((1, D))],
    out_specs=[_row_spec(D), _row_spec(1)],
    out_shape=[jax.ShapeDtypeStruct((PN, D), jnp.float32),
               jax.ShapeDtypeStruct((PN, 1), jnp.float32)],
)

_tc3 = pl.pallas_call(
    _tc3_body,
    grid=(N // BR,),
    in_specs=[_row_spec(D),
              _full_spec((D, D_OUT)), _full_spec((1, D_OUT)),
              _full_spec((D, D_OUT)), _full_spec((1, D_OUT))],
    out_specs=[_row_spec(D_OUT), _row_spec(D_OUT)],
    out_shape=[jax.ShapeDtypeStruct((N, D_OUT), jnp.float32),
               jax.ShapeDtypeStruct((N, D_OUT), jnp.float32)],
)


@jax.jit
def kernel(x, edge_index, W1, b1, W_mu, b_mu, W_logvar, b_logvar):
    ei = edge_index.astype(jnp.int32)
    npad = EPAD - E
    src = jnp.concatenate([ei[0], jnp.zeros((npad,), jnp.int32)])
    dst = jnp.concatenate([ei[1], jnp.full((npad,), PN - 1, jnp.int32)])

    xw = _tc0(x, W1)

    t0 = jnp.ones((PN, D), jnp.float32)
    dis0 = jnp.ones((PN, 1), jnp.float32)
    zerosD = jnp.zeros((1, D), jnp.float32)
    onesD = jnp.ones((1, D), jnp.float32)

    niter = 3 + lax.optimization_barrier(jnp.int32(0))

    def layer(i, carry):
        t, dis = carry
        fa = jnp.where(i == 0, onesD, zerosD)
        fb = jnp.where(i == 1, onesD, zerosD)
        b_i = jnp.where(i == 1, b1.reshape(1, D), zerosD)
        p = _sc_prop(t, src, dst).reshape(NC, PN, D)
        t2, dis2 = _tc_mid(p, t, dis, xw, b_i, fa, fb)
        return (t2, dis2)

    hp, _ = lax.fori_loop(0, niter, layer, (t0, dis0))

    mu, logvar = _tc3(hp,
                      W_mu, b_mu.reshape(1, D_OUT),
                      W_logvar, b_logvar.reshape(1, D_OUT))
    return (mu, logvar)


# speedup vs baseline: 2.4479x; 1.1268x over previous
import functools

import jax
import jax.numpy as jnp
from jax import lax
from jax.experimental import pallas as pl
from jax.experimental.pallas import tpu as pltpu
from jax.experimental.pallas import tpu_sc as plsc

N = 10000
E = 320000
D = 128
D_OUT = 64

NC = 2
NS = 16
NW = NC * NS
CH = 80
EPW = 125 * CH
EPAD = NW * EPW + CH
PN = 10240
RP = PN // NS

_sc_mesh = plsc.VectorSubcoreMesh(core_axis_name="c", subcore_axis_name="s")


@functools.partial(
    pl.kernel,
    out_type=jax.ShapeDtypeStruct((NC * PN, D), jnp.float32),
    mesh=_sc_mesh,
    scratch_types=[
        pltpu.VMEM((CH, D), jnp.float32),
        pltpu.VMEM((CH, D), jnp.float32),
        pltpu.VMEM((CH,), jnp.int32),
        pltpu.VMEM((CH,), jnp.int32),
        pltpu.VMEM((1, CH), jnp.int32),
        pltpu.VMEM((1, CH), jnp.int32),
        pltpu.SemaphoreType.DMA,
        pltpu.SemaphoreType.DMA,
        pltpu.SemaphoreType.DMA,
        pltpu.VMEM((16,), jnp.float32),
        pltpu.VMEM_SHARED((PN, D), jnp.float32),
    ],
)
def _sc_prop(t_hbm, src_hbm, dst_hbm, flag_hbm, ones_hbm, out_hbm,
             rows0, rows1, sidx0, sidx1, didx0, didx1,
             gsem, ssem, isem, fbuf, acc):
    c = lax.axis_index("c")
    s = lax.axis_index("s")
    wid = c * NS + s
    ebase = wid * EPW

    pltpu.sync_copy(flag_hbm, fbuf)
    deg_mode = fbuf[...][0] > 0.5

    def e0_of(j):
        return pl.multiple_of(ebase + j * CH, 8)

    def idx_start(j, sb, db):
        pltpu.async_copy(src_hbm.at[pl.ds(e0_of(j), CH)], sb, isem)
        pltpu.async_copy(dst_hbm.at[pl.ds(e0_of(j), CH)], db.at[0], isem)

    def idx_wait(j, sb, db):
        pltpu.make_async_copy(src_hbm.at[pl.ds(e0_of(j), CH)], sb, isem).wait()
        pltpu.make_async_copy(dst_hbm.at[pl.ds(e0_of(j), CH)], db.at[0],
                              isem).wait()

    def gather_start(sb, rw):
        pltpu.async_copy(t_hbm.at[sb], rw, gsem)

    def gather_wait(sb, rw):
        pltpu.make_async_copy(t_hbm.at[sb], rw, gsem).wait()

    def scat_start(rw, db):
        pltpu.async_copy(rw, acc.at[db.at[0]], ssem, add=True)

    def scat_wait(rw, db):
        pltpu.make_async_copy(rw, acc.at[db.at[0]], ssem).wait()

    def initc(j, carry):
        r0 = pl.multiple_of(s * RP + j * CH, 8)
        pltpu.sync_copy(t_hbm.at[pl.ds(r0, CH)], rows0)
        pltpu.sync_copy(rows0, acc.at[pl.ds(r0, CH)])
        return carry

    lax.fori_loop(0, RP // CH, initc, 0)
    plsc.subcore_barrier()

    NPAIR = (EPW // CH - 1) // 2

    @pl.when(jnp.logical_not(deg_mode))
    def _gather_arm():
        idx_start(0, sidx0, didx0)
        idx_wait(0, sidx0, didx0)
        gather_start(sidx0, rows0)
        idx_start(1, sidx1, didx1)

        def pair(k, carry):
            j = k * 2
            gather_wait(sidx0, rows0)
            scat_start(rows0, didx0)
            idx_wait(j + 1, sidx1, didx1)
            gather_start(sidx1, rows1)
            scat_wait(rows0, didx0)
            idx_start(j + 2, sidx0, didx0)
            gather_wait(sidx1, rows1)
            scat_start(rows1, didx1)
            idx_wait(j + 2, sidx0, didx0)
            gather_start(sidx0, rows0)
            scat_wait(rows1, didx1)
            idx_start(j + 3, sidx1, didx1)
            return carry

        lax.fori_loop(0, NPAIR, pair, 0)

        gather_wait(sidx0, rows0)
        scat_start(rows0, didx0)
        idx_wait(2 * NPAIR + 1, sidx1, didx1)
        scat_wait(rows0, didx0)

    @pl.when(deg_mode)
    def _deg_arm():
        pltpu.sync_copy(ones_hbm, rows0)

        def dstart(j, db):
            pltpu.async_copy(dst_hbm.at[pl.ds(e0_of(j), CH)], db.at[0], isem)

        def dwait(j, db):
            pltpu.make_async_copy(dst_hbm.at[pl.ds(e0_of(j), CH)], db.at[0],
                                  isem).wait()

        dstart(0, didx0)
        dstart(1, didx1)

        def dpair(k, carry):
            j = k * 2
            dwait(j, didx0)
            scat_start(rows0, didx0)
            dwait(j + 1, didx1)
            scat_wait(rows0, didx0)
            dstart(j + 2, didx0)
            scat_start(rows0, didx1)
            scat_wait(rows0, didx1)
            dstart(j + 3, didx1)
            return carry

        lax.fori_loop(0, NPAIR, dpair, 0)

        dwait(2 * NPAIR, didx0)
        scat_start(rows0, didx0)
        dwait(2 * NPAIR + 1, didx1)
        scat_wait(rows0, didx0)

    plsc.subcore_barrier()

    def wbc(j, carry):
        r0 = pl.multiple_of(s * RP + j * CH, 8)
        pltpu.sync_copy(acc.at[pl.ds(r0, CH)], rows0)
        pltpu.sync_copy(rows0, out_hbm.at[pl.ds(c * PN + r0, CH)])
        return carry

    lax.fori_loop(0, RP // CH, wbc, 0)


BR = 1000


def _tc0_body(x_ref, w_ref, xw_ref):
    xw_ref[...] = jnp.dot(x_ref[...], w_ref[...],
                          preferred_element_type=jnp.float32)


def _tc_mid_body(p_ref, t_ref, dis_ref, xw_ref, b_ref, fa_ref, fb_ref,
                 o_ref, dout_ref):
    comb = p_ref[0] + p_ref[1] - t_ref[...]
    is_first = fa_ref[...] > 0.0
    relu_on = fb_ref[...] > 0.0
    d = jnp.where(is_first[:, 0:1], lax.rsqrt(comb[:, 0:1]), dis_ref[...])
    zc = d * comb + b_ref[...]
    g = jnp.where(relu_on, jnp.maximum(zc, 0.0), zc)
    o_ref[...] = jnp.where(is_first, d * xw_ref[...],
                           jnp.where(relu_on, d * g, g))
    dout_ref[...] = d


def _tc3_body(hp_ref, wm_ref, bm_ref, wl_ref, bl_ref, mu_ref, lv_ref):
    hp = hp_ref[...]
    mu_ref[...] = jnp.dot(hp, wm_ref[...],
                          preferred_element_type=jnp.float32) + bm_ref[...]
    lv_ref[...] = jnp.dot(hp, wl_ref[...],
                          preferred_element_type=jnp.float32) + bl_ref[...]


def _row_spec(width):
    return pl.BlockSpec((BR, width), lambda i: (i, 0))


_pq_spec = pl.BlockSpec((2, BR, D), lambda i: (0, i, 0))


def _full_spec(shape):
    nd = len(shape)
    return pl.BlockSpec(shape, lambda i: (0,) * nd)


_tc0 = pl.pallas_call(
    _tc0_body,
    grid=(N // BR,),
    in_specs=[_row_spec(D), _full_spec((D, D))],
    out_specs=_row_spec(D),
    out_shape=jax.ShapeDtypeStruct((PN, D), jnp.float32),
)

_tc_mid = pl.pallas_call(
    _tc_mid_body,
    grid=(N // BR,),
    in_specs=[_pq_spec, _row_spec(D), _row_spec(1), _row_spec(D),
              _full_spec((1, D)), _full_spec((1, D)), _full_spec((1, D))],
    out_specs=[_row_spec(D), _row_spec(1)],
    out_shape=[jax.ShapeDtypeStruct((PN, D), jnp.float32),
               jax.ShapeDtypeStruct((PN, 1), jnp.float32)],
)

_tc3 = pl.pallas_call(
    _tc3_body,
    grid=(N // BR,),
    in_specs=[_row_spec(D),
              _full_spec((D, D_OUT)), _full_spec((1, D_OUT)),
              _full_spec((D, D_OUT)), _full_spec((1, D_OUT))],
    out_specs=[_row_spec(D_OUT), _row_spec(D_OUT)],
    out_shape=[jax.ShapeDtypeStruct((N, D_OUT), jnp.float32),
               jax.ShapeDtypeStruct((N, D_OUT), jnp.float32)],
)


@jax.jit
def kernel(x, edge_index, W1, b1, W_mu, b_mu, W_logvar, b_logvar):
    ei = edge_index.astype(jnp.int32)
    npad = EPAD - E
    src = jnp.concatenate([ei[0], jnp.zeros((npad,), jnp.int32)])
    dst = jnp.concatenate([ei[1], jnp.full((npad,), PN - 1, jnp.int32)])

    xw = _tc0(x, W1)

    t0 = jnp.ones((PN, D), jnp.float32)
    dis0 = jnp.ones((PN, 1), jnp.float32)
    zerosD = jnp.zeros((1, D), jnp.float32)
    onesD = jnp.ones((1, D), jnp.float32)

    niter = 3 + lax.optimization_barrier(jnp.int32(0))

    ones_rows = jnp.ones((CH, D), jnp.float32)
    ones16 = jnp.ones((16,), jnp.float32)
    zeros16 = jnp.zeros((16,), jnp.float32)

    def layer(i, carry):
        t, dis = carry
        fa = jnp.where(i == 0, onesD, zerosD)
        fb = jnp.where(i == 1, onesD, zerosD)
        b_i = jnp.where(i == 1, b1.reshape(1, D), zerosD)
        flag_sc = jnp.where(i == 0, ones16, zeros16)
        p = _sc_prop(t, src, dst, flag_sc, ones_rows).reshape(NC, PN, D)
        t2, dis2 = _tc_mid(p, t, dis, xw, b_i, fa, fb)
        return (t2, dis2)

    hp, _ = lax.fori_loop(0, niter, layer, (t0, dis0))

    mu, logvar = _tc3(hp,
                      W_mu, b_mu.reshape(1, D_OUT),
                      W_logvar, b_logvar.reshape(1, D_OUT))
    return (mu, logvar)
